# Initial kernel scaffold; baseline (speedup 1.0000x reference)
import jax
import jax.numpy as jnp
from jax.experimental import pallas as pl


def kernel(means2d, conics, colors, opacities, depths):
    raise NotImplementedError("write your pallas kernel here")


# speedup vs baseline: 8.1060x; 8.1060x over previous
import functools

import jax
import jax.numpy as jnp
from jax.experimental import pallas as pl
from jax.experimental.pallas import tpu as pltpu

H = 64
W = 64
G = 5000
FRONT_K = 8
ALPHA_THRESHOLD = 1.0 / 255.0

CHUNK = 256
PIXB = 512
G_PAD = ((G + CHUNK - 1) // CHUNK) * CHUNK
NCHUNK = G_PAD // CHUNK
NPIX = H * W
NBLK = NPIX // PIXB


def _raster_kernel(mx_ref, my_ref, ca_ref, cb_ref, cc_ref, op_ref,
                   col_ref, out_ref):
    pid = pl.program_id(0)
    r = jax.lax.broadcasted_iota(jnp.int32, (PIXB, 1), 0)
    p = pid * PIXB + r
    px = (p % W).astype(jnp.float32) + 0.5
    py = (p // W).astype(jnp.float32) + 0.5

    rowi = jax.lax.broadcasted_iota(jnp.int32, (CHUNK, CHUNK), 0)
    coli = jax.lax.broadcasted_iota(jnp.int32, (CHUNK, CHUNK), 1)
    tri = (rowi <= coli).astype(jnp.float32)

    def body(carry):
        i, count, T, rgb = carry
        mx = mx_ref[pl.ds(i, 1)].reshape(1, CHUNK)
        my = my_ref[pl.ds(i, 1)].reshape(1, CHUNK)
        ca = ca_ref[pl.ds(i, 1)].reshape(1, CHUNK)
        cb = cb_ref[pl.ds(i, 1)].reshape(1, CHUNK)
        cc = cc_ref[pl.ds(i, 1)].reshape(1, CHUNK)
        op = op_ref[pl.ds(i, 1)].reshape(1, CHUNK)
        cols = col_ref[pl.ds(i, 1)].reshape(CHUNK, 8)

        dx = px - mx
        dy = py - my
        power = 0.5 * (ca * dx * dx + cc * dy * dy) + cb * dx * dy
        alpha = jnp.minimum(0.999, op * jnp.exp(-power))
        visible = (alpha > ALPHA_THRESHOLD) & (power >= 0.0)
        vis_f = visible.astype(jnp.float32)
        rank_in = jax.lax.dot_general(
            vis_f, tri, (((1,), (0,)), ((), ())),
            preferred_element_type=jnp.float32)
        keep = visible & ((count + rank_in) <= float(FRONT_K))
        eff = jnp.where(keep, alpha, 0.0)
        log_t = jnp.log1p(-eff)
        cum = jax.lax.dot_general(
            log_t, tri, (((1,), (0,)), ((), ())),
            preferred_element_type=jnp.float32)
        w = eff * T * jnp.exp(cum - log_t)
        rgb = rgb + jax.lax.dot_general(
            w, cols, (((1,), (0,)), ((), ())),
            preferred_element_type=jnp.float32)
        T = T * jnp.exp(cum[:, CHUNK - 1:CHUNK])
        count = count + rank_in[:, CHUNK - 1:CHUNK]
        return i + 1, count, T, rgb

    def cond(carry):
        i, count, _, _ = carry
        return (i < NCHUNK) & (jnp.min(count) < float(FRONT_K))

    init = (jnp.int32(0),
            jnp.zeros((PIXB, 1), jnp.float32),
            jnp.ones((PIXB, 1), jnp.float32),
            jnp.zeros((PIXB, 8), jnp.float32))
    _, _, _, rgb = jax.lax.while_loop(cond, body, init)
    out_ref[...] = rgb


@functools.partial(jax.jit, static_argnames=())
def _run(means2d, conics, colors, opacities, depths):
    order = jnp.argsort(depths[0], stable=True)
    means_s = means2d[0][order]
    conics_s = conics[0][order]
    colors_s = colors[0][order]
    opac_s = opacities[0][order]

    pad = G_PAD - G

    def chunked(x):
        x = jnp.pad(x, (0, pad))
        return x.reshape(NCHUNK, 1, CHUNK)

    mx = chunked(means_s[:, 0])
    my = chunked(means_s[:, 1])
    ca = chunked(conics_s[:, 0])
    cb = chunked(conics_s[:, 1])
    cc = chunked(conics_s[:, 2])
    op = chunked(opac_s)
    cols = jnp.pad(colors_s, ((0, pad), (0, 5))).reshape(NCHUNK, CHUNK, 8)

    full = lambda s: pl.BlockSpec(s, lambda i: (0,) * len(s))
    out = pl.pallas_call(
        _raster_kernel,
        grid=(NBLK,),
        in_specs=[
            full((NCHUNK, 1, CHUNK)), full((NCHUNK, 1, CHUNK)),
            full((NCHUNK, 1, CHUNK)), full((NCHUNK, 1, CHUNK)),
            full((NCHUNK, 1, CHUNK)), full((NCHUNK, 1, CHUNK)),
            full((NCHUNK, CHUNK, 8)),
        ],
        out_specs=pl.BlockSpec((PIXB, 8), lambda i: (i, 0)),
        out_shape=jax.ShapeDtypeStruct((NPIX, 8), jnp.float32),
        compiler_params=pltpu.CompilerParams(
            dimension_semantics=("parallel",)),
    )(mx, my, ca, cb, cc, op, cols)
    return out[:, :3].reshape(1, H, W, 3)


def kernel(means2d, conics, colors, opacities, depths):
    return _run(means2d, conics, colors, opacities, depths)


# speedup vs baseline: 10.8896x; 1.3434x over previous
import functools

import jax
import jax.numpy as jnp
from jax.experimental import pallas as pl
from jax.experimental.pallas import tpu as pltpu

H = 64
W = 64
G = 5000
FRONT_K = 8
ALPHA_THRESHOLD = 1.0 / 255.0

CHUNK = 256
PIXB = 512
G_PAD = ((G + CHUNK - 1) // CHUNK) * CHUNK
NCHUNK = G_PAD // CHUNK
NPIX = H * W
NBLK = NPIX // PIXB


def _raster_kernel(mx_ref, my_ref, ca_ref, cb_ref, cc_ref, op_ref,
                   col_ref, out_ref):
    pid = pl.program_id(0)
    r = jax.lax.broadcasted_iota(jnp.int32, (PIXB, 1), 0)
    p = pid * PIXB + r
    px = (p % W).astype(jnp.float32) + 0.5
    py = (p // W).astype(jnp.float32) + 0.5

    rowi = jax.lax.broadcasted_iota(jnp.int32, (CHUNK, CHUNK), 0)
    coli = jax.lax.broadcasted_iota(jnp.int32, (CHUNK, CHUNK), 1)
    tri = (rowi <= coli).astype(jnp.float32)

    def body(carry):
        i, count, T, rgb = carry
        mx = mx_ref[pl.ds(i, 1)].reshape(1, CHUNK)
        my = my_ref[pl.ds(i, 1)].reshape(1, CHUNK)
        ca = ca_ref[pl.ds(i, 1)].reshape(1, CHUNK)
        cb = cb_ref[pl.ds(i, 1)].reshape(1, CHUNK)
        cc = cc_ref[pl.ds(i, 1)].reshape(1, CHUNK)
        op = op_ref[pl.ds(i, 1)].reshape(1, CHUNK)
        cols = col_ref[pl.ds(i, 1)].reshape(CHUNK, 8)

        dx = px - mx
        dy = py - my
        power = 0.5 * (ca * dx * dx + cc * dy * dy) + cb * dx * dy
        alpha = jnp.minimum(0.999, op * jnp.exp(-power))
        visible = (alpha > ALPHA_THRESHOLD) & (power >= 0.0)
        vis_f = visible.astype(jnp.float32)
        rank_in = jax.lax.dot_general(
            vis_f, tri, (((1,), (0,)), ((), ())),
            preferred_element_type=jnp.float32)
        keep = visible & ((count + rank_in) <= float(FRONT_K))
        eff = jnp.where(keep, alpha, 0.0)
        log_t = jnp.log1p(-eff)
        cum = jax.lax.dot_general(
            log_t, tri, (((1,), (0,)), ((), ())),
            preferred_element_type=jnp.float32)
        w = eff * T * jnp.exp(cum - log_t)
        rgb = rgb + jax.lax.dot_general(
            w, cols, (((1,), (0,)), ((), ())),
            preferred_element_type=jnp.float32)
        T = T * jnp.exp(cum[:, CHUNK - 1:CHUNK])
        count = count + rank_in[:, CHUNK - 1:CHUNK]
        return i + 1, count, T, rgb

    def cond(carry):
        i, count, _, _ = carry
        return (i < NCHUNK) & (jnp.min(count) < float(FRONT_K))

    init = (jnp.int32(0),
            jnp.zeros((PIXB, 1), jnp.float32),
            jnp.ones((PIXB, 1), jnp.float32),
            jnp.zeros((PIXB, 8), jnp.float32))
    _, _, _, rgb = jax.lax.while_loop(cond, body, init)
    out_ref[...] = rgb


@functools.partial(jax.jit, static_argnames=())
def _run(means2d, conics, colors, opacities, depths):
    order = jnp.argsort(depths[0], stable=True)
    packed = jnp.concatenate(
        [means2d[0], conics[0], opacities[0][:, None], colors[0]], axis=1)
    packed_s = packed[order]
    pad = G_PAD - G
    packed_s = jnp.pad(packed_s, ((0, pad), (0, 0)))
    attr_t = packed_s.T

    def chunked(row):
        return attr_t[row].reshape(NCHUNK, 1, CHUNK)

    mx = chunked(0)
    my = chunked(1)
    ca = chunked(2)
    cb = chunked(3)
    cc = chunked(4)
    op = chunked(5)
    cols = jnp.pad(packed_s[:, 6:9], ((0, 0), (0, 5))).reshape(
        NCHUNK, CHUNK, 8)

    full = lambda s: pl.BlockSpec(s, lambda i: (0,) * len(s))
    out = pl.pallas_call(
        _raster_kernel,
        grid=(NBLK,),
        in_specs=[
            full((NCHUNK, 1, CHUNK)), full((NCHUNK, 1, CHUNK)),
            full((NCHUNK, 1, CHUNK)), full((NCHUNK, 1, CHUNK)),
            full((NCHUNK, 1, CHUNK)), full((NCHUNK, 1, CHUNK)),
            full((NCHUNK, CHUNK, 8)),
        ],
        out_specs=pl.BlockSpec((PIXB, 8), lambda i: (i, 0)),
        out_shape=jax.ShapeDtypeStruct((NPIX, 8), jnp.float32),
        compiler_params=pltpu.CompilerParams(
            dimension_semantics=("parallel",)),
    )(mx, my, ca, cb, cc, op, cols)
    return out[:, :3].reshape(1, H, W, 3)


def kernel(means2d, conics, colors, opacities, depths):
    return _run(means2d, conics, colors, opacities, depths)
